# Initial kernel scaffold; baseline (speedup 1.0000x reference)
import jax
import jax.numpy as jnp
from jax.experimental import pallas as pl


def kernel(teacher_input, W1, b1, W2, b2, edge_row, edge_col):
    raise NotImplementedError("write your pallas kernel here")


# speedup vs baseline: 14.1022x; 14.1022x over previous
import functools

import jax
import jax.numpy as jnp
from jax import lax
from jax.experimental import pallas as pl
from jax.experimental.pallas import tpu as pltpu
from jax.experimental.pallas import tpu_sc as plsc

N_U = 50000
N_I = 50000
N_E = 800000
D = 64

PAD_N = 50176
HALF = PAD_N // 2
STRIPE = HALF // 16
ACC_ROWS = HALF + 16
DUMMY = HALF
C = 128
BM = 1024


def _mlp_norm_body(x_ref, w1_ref, b1_ref, w2_ref, b2_ref, o_ref):
    x = x_ref[...]
    h = jnp.dot(x, w1_ref[...], preferred_element_type=jnp.float32)
    h = jnp.maximum(h + b1_ref[...], 0.0)
    f = jnp.dot(h, w2_ref[...], preferred_element_type=jnp.float32)
    f = f + b2_ref[...]
    n = jnp.sqrt(jnp.sum(f * f, axis=-1, keepdims=True))
    o_ref[...] = f / jnp.maximum(n, 1e-12)


def _l2norm_body(x_ref, o_ref):
    x = x_ref[...]
    n = jnp.sqrt(jnp.sum(x * x, axis=-1, keepdims=True))
    o_ref[...] = x / jnp.maximum(n, 1e-12)


def _mlp_norm(x, W1, b1, W2, b2):
    k_in = W1.shape[0]
    hid = W1.shape[1]
    return pl.pallas_call(
        _mlp_norm_body,
        grid=(PAD_N // BM,),
        in_specs=[
            pl.BlockSpec((BM, k_in), lambda i: (i, 0)),
            pl.BlockSpec((k_in, hid), lambda i: (0, 0)),
            pl.BlockSpec((1, hid), lambda i: (0, 0)),
            pl.BlockSpec((hid, D), lambda i: (0, 0)),
            pl.BlockSpec((1, D), lambda i: (0, 0)),
        ],
        out_specs=pl.BlockSpec((BM, D), lambda i: (i, 0)),
        out_shape=jax.ShapeDtypeStruct((PAD_N, D), jnp.float32),
    )(x, W1, b1.reshape(1, -1), W2, b2.reshape(1, -1))


def _l2norm(x):
    return pl.pallas_call(
        _l2norm_body,
        grid=(PAD_N // BM,),
        in_specs=[pl.BlockSpec((BM, D), lambda i: (i, 0))],
        out_specs=pl.BlockSpec((BM, D), lambda i: (i, 0)),
        out_shape=jax.ShapeDtypeStruct((PAD_N, D), jnp.float32),
    )(x)


def _spmm_body(fn_hbm, col_hbm, row_hbm, bounds_hbm, out_hbm,
               bounds_v, col_v, loc_v, row_v, rows_v, zbuf_v, acc_sh, sem):
    c = lax.axis_index("c")
    s = lax.axis_index("s")
    w = c * 16 + s

    def _zb(r, carry):
        for j in range(D // 16):
            zbuf_v[r, pl.ds(j * 16, 16)] = jnp.zeros((16,), jnp.float32)
        return carry
    lax.fori_loop(0, C, _zb, 0)
    base_r = s * STRIPE
    for k in range(STRIPE // C):
        pltpu.sync_copy(zbuf_v, acc_sh.at[pl.ds(base_r + k * C, C)])
    rem = STRIPE % C
    if rem:
        pltpu.sync_copy(zbuf_v.at[pl.ds(0, rem)],
                        acc_sh.at[pl.ds(base_r + (STRIPE // C) * C, rem)])
    plsc.subcore_barrier()

    pltpu.sync_copy(bounds_hbm, bounds_v)
    iota = lax.iota(jnp.int32, 16)

    e_start = bounds_v[pl.ds(w, 16)][0]
    e_end = bounds_v[pl.ds(32 + w, 16)][0]
    e0 = (e_start // 8) * 8
    nch = (e_end - e0 + (C - 1)) // C
    sc_base = c * HALF

    def _chunk(i, carry):
        base = e0 + i * C
        pltpu.sync_copy(col_hbm.at[pl.ds(base, C)], col_v)
        pltpu.sync_copy(row_hbm.at[pl.ds(base, C)], row_v)

        def _fix(j, cc):
            eid = base + j * 16 + iota
            m = (eid >= e_start) & (eid < e_end)
            cv = col_v[pl.ds(j * 16, 16)]
            rv = row_v[pl.ds(j * 16, 16)]
            col_v[pl.ds(j * 16, 16)] = jnp.where(m, cv, 0)
            loc_v[pl.ds(j * 16, 16)] = jnp.where(m, rv - sc_base, DUMMY)
            return cc
        lax.fori_loop(0, C // 16, _fix, 0)

        pltpu.async_copy(fn_hbm.at[col_v], rows_v, sem).wait()
        pltpu.sync_copy(rows_v, acc_sh.at[loc_v], add=True)
        return carry
    lax.fori_loop(0, nch, _chunk, 0)
    plsc.subcore_barrier()

    pltpu.sync_copy(acc_sh.at[pl.ds(base_r, STRIPE)],
                    out_hbm.at[pl.ds(sc_base + base_r, STRIPE)])


_spmm = pl.kernel(
    _spmm_body,
    out_type=jax.ShapeDtypeStruct((PAD_N, D), jnp.float32),
    mesh=plsc.VectorSubcoreMesh(core_axis_name="c", subcore_axis_name="s"),
    compiler_params=pltpu.CompilerParams(use_tc_tiling_on_sc=False),
    scratch_types=[
        pltpu.VMEM((80,), jnp.int32),
        pltpu.VMEM((C,), jnp.int32),
        pltpu.VMEM((C,), jnp.int32),
        pltpu.VMEM((C,), jnp.int32),
        pltpu.VMEM((C, D), jnp.float32),
        pltpu.VMEM((C, D), jnp.float32),
        pltpu.VMEM_SHARED((ACC_ROWS, D), jnp.float32),
        pltpu.SemaphoreType.DMA,
    ],
)


def kernel(teacher_input, W1, b1, W2, b2, edge_row, edge_col):
    x = jnp.pad(teacher_input, ((0, PAD_N - N_I), (0, 0)))
    fn_p = _mlp_norm(x, W1, b1, W2, b2)

    em = jnp.searchsorted(edge_row, HALF).astype(jnp.int32)
    t = jnp.arange(16, dtype=jnp.int32)
    sz0 = (em + 15) // 16
    sz1 = (N_E - em + 15) // 16
    s0 = jnp.minimum(t * sz0, em)
    e0 = jnp.minimum(s0 + sz0, em)
    s1 = jnp.minimum(em + t * sz1, N_E)
    e1 = jnp.minimum(s1 + sz1, N_E)
    bounds = jnp.concatenate(
        [s0, s1, e0, e1, jnp.zeros((16,), jnp.int32)]).astype(jnp.int32)

    colp = jnp.pad(edge_col, (0, C))
    rowp = jnp.pad(edge_row, (0, C))

    uv = _spmm(fn_p, colp, rowp, bounds)
    out1 = _l2norm(uv)
    return out1[:N_U], fn_p[:N_I]
